# baseline (speedup 1.0000x reference)
import functools

import jax
import jax.numpy as jnp
from jax import lax
from jax.experimental import pallas as pl
from jax.experimental.pallas import tpu as pltpu
from jax.experimental.pallas import tpu_sc as plsc

B = 16384
DU = 32
DI = 64
NH = 64

_info = plsc.get_sparse_core_info()
NC = _info.num_cores
NS = _info.num_subcores
NW = NC * NS
BPW = B // NW
CHUNK = 128
NCH = BPW // CHUNK

_mesh = plsc.VectorSubcoreMesh(core_axis_name="c", subcore_axis_name="s")


@functools.partial(
    pl.kernel,
    mesh=_mesh,
    out_type=[
        jax.ShapeDtypeStruct((B, DU), jnp.float32),
        jax.ShapeDtypeStruct((B, DI), jnp.float32),
    ],
    scratch_types=[
        pltpu.VMEM((NCH, CHUNK), jnp.int32),
        pltpu.VMEM((NCH, CHUNK), jnp.int32),
        pltpu.VMEM((BPW, DU), jnp.float32),
        pltpu.VMEM((BPW, DI), jnp.float32),
        pltpu.SemaphoreType.DMA,
        pltpu.SemaphoreType.DMA,
    ],
    compiler_params=pltpu.CompilerParams(use_tc_tiling_on_sc=False),
)
def _sc_gather(idx_u_hbm, idx_i_hbm, emb_u_hbm, emb_i_hbm,
               hu_hbm, hi_hbm,
               idx_u_v, idx_i_v, rows_u, rows_i, sem_u, sem_i):
    wid = lax.axis_index("s") * NC + lax.axis_index("c")
    base = wid * BPW
    pltpu.sync_copy(idx_u_hbm.at[wid], idx_u_v)
    pltpu.sync_copy(idx_i_hbm.at[wid], idx_i_v)
    copies = []
    for j in range(NCH):
        copies.append(pltpu.async_copy(
            emb_u_hbm.at[idx_u_v.at[j]],
            rows_u.at[pl.ds(j * CHUNK, CHUNK)], sem_u))
        copies.append(pltpu.async_copy(
            emb_i_hbm.at[idx_i_v.at[j]],
            rows_i.at[pl.ds(j * CHUNK, CHUNK)], sem_i))
    for c in copies:
        c.wait()
    pltpu.sync_copy(rows_u, hu_hbm.at[pl.ds(base, BPW)])
    pltpu.sync_copy(rows_i, hi_hbm.at[pl.ds(base, BPW)])


def _tc_proj(hu_ref, hi_ref, wu_ref, bu_ref, wi_ref, bi_ref, ou_ref, oi_ref):
    ou_ref[...] = jnp.maximum(
        jnp.dot(hu_ref[...], wu_ref[...],
                preferred_element_type=jnp.float32) + bu_ref[...], 0.0)
    oi_ref[...] = jnp.maximum(
        jnp.dot(hi_ref[...], wi_ref[...],
                preferred_element_type=jnp.float32) + bi_ref[...], 0.0)


RB = 2048


def kernel(idx_user, idx_item, emb_user, emb_item, W_user, b_user, W_item, b_item):
    idx_u = idx_user.astype(jnp.int32).reshape(NW, NCH, CHUNK)
    idx_i = idx_item.astype(jnp.int32).reshape(NW, NCH, CHUNK)
    hu, hi = _sc_gather(idx_u, idx_i, emb_user, emb_item)

    grid = (B // RB,)
    ou, oi = pl.pallas_call(
        _tc_proj,
        grid=grid,
        in_specs=[
            pl.BlockSpec((RB, DU), lambda i: (i, 0)),
            pl.BlockSpec((RB, DI), lambda i: (i, 0)),
            pl.BlockSpec((DU, NH), lambda i: (0, 0)),
            pl.BlockSpec((1, NH), lambda i: (0, 0)),
            pl.BlockSpec((DI, NH), lambda i: (0, 0)),
            pl.BlockSpec((1, NH), lambda i: (0, 0)),
        ],
        out_specs=[
            pl.BlockSpec((RB, NH), lambda i: (i, 0)),
            pl.BlockSpec((RB, NH), lambda i: (i, 0)),
        ],
        out_shape=[
            jax.ShapeDtypeStruct((B, NH), jnp.float32),
            jax.ShapeDtypeStruct((B, NH), jnp.float32),
        ],
    )(hu, hi, W_user, b_user.reshape(1, NH), W_item, b_item.reshape(1, NH))
    return (ou, oi)
